# baseline (speedup 1.0000x reference)
import jax
import jax.numpy as jnp
from jax.experimental import pallas as pl
from jax.experimental.pallas import tpu as pltpu

T = 2048
D = 2048
E = 16
CAP = 128
BT = 256
NB = T // BT
BR = BT * E


def _gate_kernel(x_ref, w_ref, laux_ref, cnt_ref, a_ref, g_ref,
                 me_acc, carry):
    i = pl.program_id(0)

    @pl.when(i == 0)
    def _init():
        me_acc[...] = jnp.zeros_like(me_acc)
        carry[...] = jnp.zeros_like(carry)

    x = x_ref[...]
    w = w_ref[...]
    logits = jax.lax.dot_general(
        x, w, (((1,), (1,)), ((), ())),
        preferred_element_type=jnp.float32)

    m = jnp.max(logits, axis=1, keepdims=True)
    p = jnp.exp(logits - m)
    s = jnp.sum(p, axis=1, keepdims=True)
    gates = p / s
    gmax = jnp.max(gates, axis=1, keepdims=True)

    colid = jax.lax.broadcasted_iota(jnp.int32, (BT, E), 1)
    idx = jnp.min(jnp.where(gates == gmax, colid, E),
                  axis=1, keepdims=True)
    onehot = colid == idx
    mask1 = onehot.astype(jnp.float32)

    me_acc[...] += jnp.sum(gates, axis=0, keepdims=True)

    r = jax.lax.broadcasted_iota(jnp.int32, (BT, BT), 0)
    c = jax.lax.broadcasted_iota(jnp.int32, (BT, BT), 1)
    tri = (r >= c).astype(jnp.float32)
    incl = carry[...] + jax.lax.dot_general(
        tri, mask1, (((1,), (0,)), ((), ())),
        preferred_element_type=jnp.float32)
    carry[...] = incl[BT - 1:BT, :]

    a_ref[...] = jnp.where(onehot, incl - 1.0, -1.0)
    g_ref[...] = jnp.where(onehot, gmax, 0.0)

    @pl.when(i == NB - 1)
    def _fin():
        cnt = carry[...]
        cnt_ref[...] = cnt.astype(jnp.int32)
        laux_ref[...] = jnp.sum(me_acc[...] * cnt, axis=1,
                                keepdims=True) * (E / (T * T))


def _expand_kernel(a_ref, g_ref, cw_ref, disp_ref, p_ref):
    i = pl.program_id(0)

    @pl.when(i == 0)
    def _init():
        rr = jax.lax.broadcasted_iota(jnp.int32, (BR, BT), 0)
        tt = jax.lax.broadcasted_iota(jnp.int32, (BR, BT), 1)
        p_ref[...] = (rr // E == tt).astype(jnp.float32)

    da = jax.lax.dot_general(
        p_ref[...], a_ref[...], (((1,), (0,)), ((), ())),
        preferred_element_type=jnp.float32,
        precision=jax.lax.Precision.HIGHEST)
    dg = jax.lax.dot_general(
        p_ref[...], g_ref[...], (((1,), (0,)), ((), ())),
        preferred_element_type=jnp.float32,
        precision=jax.lax.Precision.HIGHEST)
    lane16 = jax.lax.broadcasted_iota(jnp.int32, (BR, E), 1)
    sub16 = jax.lax.broadcasted_iota(jnp.int32, (BR, E), 0) % E
    msel = lane16 == sub16
    code = jnp.sum(jnp.where(msel, da, 0.0), axis=1,
                   keepdims=True).astype(jnp.int32)
    gval = jnp.sum(jnp.where(msel, dg, 0.0), axis=1, keepdims=True)

    lane = jax.lax.broadcasted_iota(jnp.int32, (BR, CAP), 1)
    hit = lane == code
    cw_ref[...] = jnp.where(hit, gval, 0.0)
    disp_ref[...] = hit.astype(jnp.int8)


def kernel(input, wg_weight):
    laux, cnt, a, g = pl.pallas_call(
        _gate_kernel,
        grid=(NB,),
        in_specs=[
            pl.BlockSpec((BT, D), lambda i: (i, 0)),
            pl.BlockSpec((E, D), lambda i: (0, 0)),
        ],
        out_specs=[
            pl.BlockSpec((1, 1), lambda i: (0, 0)),
            pl.BlockSpec((1, E), lambda i: (0, 0)),
            pl.BlockSpec((BT, E), lambda i: (i, 0)),
            pl.BlockSpec((BT, E), lambda i: (i, 0)),
        ],
        out_shape=[
            jax.ShapeDtypeStruct((1, 1), jnp.float32),
            jax.ShapeDtypeStruct((1, E), jnp.int32),
            jax.ShapeDtypeStruct((T, E), jnp.float32),
            jax.ShapeDtypeStruct((T, E), jnp.float32),
        ],
        scratch_shapes=[
            pltpu.VMEM((1, E), jnp.float32),
            pltpu.VMEM((1, E), jnp.float32),
        ],
        compiler_params=pltpu.CompilerParams(
            dimension_semantics=("arbitrary",)),
    )(input.astype(jnp.float32), wg_weight.astype(jnp.float32))

    cw, disp = pl.pallas_call(
        _expand_kernel,
        grid=(NB,),
        in_specs=[
            pl.BlockSpec((BT, E), lambda i: (i, 0)),
            pl.BlockSpec((BT, E), lambda i: (i, 0)),
        ],
        out_specs=[
            pl.BlockSpec((BR, CAP), lambda i: (i, 0)),
            pl.BlockSpec((BR, CAP), lambda i: (i, 0)),
        ],
        out_shape=[
            jax.ShapeDtypeStruct((T * E, CAP), jnp.float32),
            jax.ShapeDtypeStruct((T * E, CAP), jnp.int8),
        ],
        scratch_shapes=[
            pltpu.VMEM((BR, BT), jnp.float32),
        ],
        compiler_params=pltpu.CompilerParams(
            dimension_semantics=("arbitrary",)),
    )(a, g)

    return (laux.reshape(()), cw.reshape(T, E, CAP),
            disp.reshape(T, E, CAP).astype(jnp.bool_), cnt.reshape(E))


# speedup vs baseline: 2.1865x; 2.1865x over previous
import jax
import jax.numpy as jnp
from jax.experimental import pallas as pl
from jax.experimental.pallas import tpu as pltpu

T = 2048
D = 2048
E = 16
CAP = 128
BT = 256
NB = T // BT
BR = BT * E


def _gate_kernel(x_ref, w_ref, laux_ref, cnt_ref, a_ref, g_ref,
                 me_acc, carry):
    i = pl.program_id(0)

    @pl.when(i == 0)
    def _init():
        me_acc[...] = jnp.zeros_like(me_acc)
        carry[...] = jnp.zeros_like(carry)

    x = x_ref[...]
    w = w_ref[...]
    logits = jax.lax.dot_general(
        x, w, (((1,), (1,)), ((), ())),
        preferred_element_type=jnp.float32)

    m = jnp.max(logits, axis=1, keepdims=True)
    p = jnp.exp(logits - m)
    s = jnp.sum(p, axis=1, keepdims=True)
    gates = p / s
    gmax = jnp.max(gates, axis=1, keepdims=True)

    colid = jax.lax.broadcasted_iota(jnp.int32, (BT, E), 1)
    idx = jnp.min(jnp.where(gates == gmax, colid, E),
                  axis=1, keepdims=True)
    onehot = colid == idx
    mask1 = onehot.astype(jnp.float32)

    me_acc[...] += jnp.sum(gates, axis=0, keepdims=True)

    r = jax.lax.broadcasted_iota(jnp.int32, (BT, BT), 0)
    c = jax.lax.broadcasted_iota(jnp.int32, (BT, BT), 1)
    tri = (r >= c).astype(jnp.float32)
    incl = carry[...] + jax.lax.dot_general(
        tri, mask1, (((1,), (0,)), ((), ())),
        preferred_element_type=jnp.float32)
    carry[...] = incl[BT - 1:BT, :]

    a_ref[...] = jnp.where(onehot, incl - 1.0, -1.0)
    g_ref[...] = jnp.where(onehot, gmax, 0.0)

    @pl.when(i == NB - 1)
    def _fin():
        cnt = carry[...]
        cnt_ref[...] = cnt.astype(jnp.int32)
        laux_ref[...] = jnp.sum(me_acc[...] * cnt, axis=1,
                                keepdims=True) * (E / (T * T))


def _expand_kernel(a_ref, g_ref, cw_ref, disp_ref):
    code = a_ref[...].astype(jnp.int32)
    sub = jax.lax.broadcasted_iota(jnp.int32, (CAP, BR), 0)
    hit_t = sub == code
    cw_t = jnp.where(hit_t, g_ref[...], 0.0)
    cw = jnp.transpose(cw_t)
    cw_ref[...] = cw
    disp_ref[...] = (cw > 0.0).astype(jnp.int8)


def kernel(input, wg_weight):
    laux, cnt, a, g = pl.pallas_call(
        _gate_kernel,
        grid=(NB,),
        in_specs=[
            pl.BlockSpec((BT, D), lambda i: (i, 0)),
            pl.BlockSpec((E, D), lambda i: (0, 0)),
        ],
        out_specs=[
            pl.BlockSpec((1, 1), lambda i: (0, 0)),
            pl.BlockSpec((1, E), lambda i: (0, 0)),
            pl.BlockSpec((BT, E), lambda i: (i, 0)),
            pl.BlockSpec((BT, E), lambda i: (i, 0)),
        ],
        out_shape=[
            jax.ShapeDtypeStruct((1, 1), jnp.float32),
            jax.ShapeDtypeStruct((1, E), jnp.int32),
            jax.ShapeDtypeStruct((T, E), jnp.float32),
            jax.ShapeDtypeStruct((T, E), jnp.float32),
        ],
        scratch_shapes=[
            pltpu.VMEM((1, E), jnp.float32),
            pltpu.VMEM((1, E), jnp.float32),
        ],
        compiler_params=pltpu.CompilerParams(
            dimension_semantics=("arbitrary",)),
    )(input.astype(jnp.float32), wg_weight.astype(jnp.float32))

    al = a.reshape(1, T * E)
    gl = g.reshape(1, T * E)

    cw, disp = pl.pallas_call(
        _expand_kernel,
        grid=(NB,),
        in_specs=[
            pl.BlockSpec((1, BR), lambda i: (0, i)),
            pl.BlockSpec((1, BR), lambda i: (0, i)),
        ],
        out_specs=[
            pl.BlockSpec((BR, CAP), lambda i: (i, 0)),
            pl.BlockSpec((BR, CAP), lambda i: (i, 0)),
        ],
        out_shape=[
            jax.ShapeDtypeStruct((T * E, CAP), jnp.float32),
            jax.ShapeDtypeStruct((T * E, CAP), jnp.int8),
        ],
    )(al, gl)

    return (laux.reshape(()), cw.reshape(T, E, CAP),
            disp.reshape(T, E, CAP).astype(jnp.bool_), cnt.reshape(E))
